# baseline (speedup 1.0000x reference)
import jax
import jax.numpy as jnp
from jax import lax
from jax.experimental import pallas as pl
from jax.experimental.pallas import tpu as pltpu
from jax.experimental.pallas import tpu_sc as plsc

_EMBED_DIM = 32
_BATCH = 4096
_SEQ_LEN = 200
_VOCAB = 1000000

_NC = 2
_NS = 16
_NW = _NC * _NS
_BB = _BATCH // _NW
_DT = _EMBED_DIM // 8
_XP = 136

_NFULL = _VOCAB // 128
_BPW = 246
_ANB = 6
_TAIL_V = _NFULL * 128


def _body_a(tv_hbm, ttail_hbm, tl_hbm, vin, vout, vtail, vtout, gsem, wsem):
    w = lax.axis_index("s") * _NC + lax.axis_index("c")

    lane = lax.iota(jnp.int32, 16)
    dv = [(lane + 16 * jg) % _EMBED_DIM for jg in range(8)]
    sv = [(lane + 16 * jg) // _EMBED_DIM for jg in range(8)]

    def c0_of(i):
        return pl.multiple_of(
            jnp.minimum(w * _BPW + i, _NFULL - 1) * 128, 128
        )

    def fire_in(i, nb):
        pltpu.async_copy(
            tv_hbm.at[:, pl.ds(c0_of(i), 128)],
            vin.at[nb].at[:, pl.ds(0, 128)],
            gsem.at[nb],
        )

    def drain_in(nb):
        pltpu.make_async_copy(
            tv_hbm.at[:, pl.ds(0, 128)],
            vin.at[nb].at[:, pl.ds(0, 128)],
            gsem.at[nb],
        ).wait()

    def fire_out(i, nb):
        pltpu.async_copy(
            vout.at[nb],
            tl_hbm.at[pl.ds(pl.multiple_of(c0_of(i) // 4, 8), 32)],
            wsem.at[nb],
        )

    def drain_out(nb):
        pltpu.make_async_copy(
            vout.at[nb], tl_hbm.at[pl.ds(0, 32)], wsem.at[nb]
        ).wait()

    def transpose(nb):
        @pl.loop(0, 32)
        def _row(r):
            for jg in range(8):
                v = plsc.load_gather(vin.at[nb], [dv[jg], sv[jg] + 4 * r])
                vout[nb, r, pl.ds(16 * jg, 16)] = v

    for b in range(_ANB):
        fire_in(b, b)
    for b in range(_ANB):
        drain_in(b)
        transpose(b)
        fire_in(b + _ANB, b)
        fire_out(b, b)

    @pl.loop(1, _BPW // _ANB - 1)
    def _grp(g):
        i0 = g * _ANB
        for b in range(_ANB):
            i = i0 + b
            drain_in(b)
            drain_out(b)
            transpose(b)
            fire_in(i + _ANB, b)
            fire_out(i, b)

    for b in range(_ANB):
        i = _BPW - _ANB + b
        drain_in(b)
        drain_out(b)
        transpose(b)
        fire_out(i, b)
    for b in range(_ANB):
        drain_out(b)

    @pl.when(w == 0)
    def _tail():
        pltpu.sync_copy(ttail_hbm, vtail)
        for r in range(16):
            for jg in range(8):
                v = plsc.load_gather(vtail, [sv[jg] + 4 * r, dv[jg]])
                vtout[r, pl.ds(16 * jg, 16)] = v
        pltpu.sync_copy(vtout, tl_hbm.at[pl.ds(_TAIL_V // 4, 16)])


def _body_b(x_hbm, table_hbm, o5_hbm, xblk, xt, rows_v, t_v, gsem, wsem):
    w = lax.axis_index("s") * _NC + lax.axis_index("c")

    pltpu.sync_copy(x_hbm.at[pl.ds(w * _BB, _BB)], xblk)

    lane = lax.iota(jnp.int32, 16)
    nh = (_SEQ_LEN + 15) // 16
    l_off = [16 * h if 16 * h + 16 <= _SEQ_LEN else _SEQ_LEN - 16
             for h in range(nh)]

    @pl.loop(0, _BB)
    def _xpose(b):
        bi = jnp.full((16,), 1, jnp.int32) * b
        for h in range(nh):
            v = xblk[b, pl.ds(l_off[h], 16)]
            plsc.store_scatter(xt, [lane + l_off[h], bi], v)

    dt_idx = [(lane + 16 * h) // 8 for h in range(2)]
    di_idx = [(lane + 16 * h) % 8 for h in range(2)]

    def fire_gather(l, nb):
        pltpu.async_copy(
            table_hbm.at[xt.at[l, pl.ds(0, _BB)]],
            rows_v.at[nb],
            gsem.at[nb],
        )

    def drain_gather(nb):
        pltpu.make_async_copy(
            table_hbm.at[pl.ds(0, _BB)], rows_v.at[nb], gsem.at[nb]
        ).wait()

    def fire_write(l, nb):
        pltpu.async_copy(
            t_v.at[nb].at[:, :, pl.ds(0, _BB)], o5_hbm.at[l].at[:, w],
            wsem.at[nb],
        )

    def drain_write(nb):
        pltpu.make_async_copy(
            t_v.at[nb].at[:, :, pl.ds(0, _BB)], o5_hbm.at[0].at[:, 0],
            wsem.at[nb],
        ).wait()

    def transpose(nb):
        @pl.loop(0, _BB)
        def _tpose(b):
            bi = jnp.full((16,), 1, jnp.int32) * b
            for h in range(2):
                v = rows_v[nb, b, pl.ds(16 * h, 16)]
                plsc.store_scatter(t_v.at[nb], [dt_idx[h], di_idx[h], bi], v)

    fire_gather(0, 0)
    fire_gather(1, 1)
    for b in range(2):
        drain_gather(b)
        transpose(b)
        fire_gather(b + 2, b)
        fire_write(b, b)

    @pl.loop(1, (_SEQ_LEN - 4) // 2 + 1)
    def _pair(p):
        l0 = 2 * p
        for b in range(2):
            l = l0 + b
            drain_gather(b)
            drain_write(b)
            transpose(b)
            fire_gather(l + 2, b)
            fire_write(l, b)

    for b in range(2):
        l = _SEQ_LEN - 2 + b
        drain_gather(b)
        drain_write(b)
        transpose(b)
        fire_write(l, b)
    for b in range(2):
        drain_write(b)


@jax.jit
def kernel(x, table):
    mesh = plsc.VectorSubcoreMesh(
        core_axis_name="c", subcore_axis_name="s",
        num_cores=_NC, num_subcores=_NS,
    )

    tlin = pl.kernel(
        _body_a,
        out_type=jax.ShapeDtypeStruct((_VOCAB // 4, 128), jnp.float32),
        mesh=mesh,
        scratch_types=[
            pltpu.VMEM((_ANB, _EMBED_DIM, 129), jnp.float32),
            pltpu.VMEM((_ANB, _EMBED_DIM, 128), jnp.float32),
            pltpu.VMEM((_VOCAB - _TAIL_V, _EMBED_DIM), jnp.float32),
            pltpu.VMEM(((_VOCAB - _TAIL_V) // 4, 128), jnp.float32),
            pltpu.SemaphoreType.DMA((2,)),
            pltpu.SemaphoreType.DMA((2,)),
        ],
        compiler_params=pltpu.CompilerParams(
            use_tc_tiling_on_sc=True, needs_layout_passes=False
        ),
    )(jnp.transpose(table),
      lax.slice(table, (_TAIL_V, 0), (_VOCAB, _EMBED_DIM)))

    o5 = pl.kernel(
        _body_b,
        out_type=jax.ShapeDtypeStruct(
            (_SEQ_LEN, _DT, _NW, 8, _BB), jnp.float32
        ),
        mesh=mesh,
        scratch_types=[
            pltpu.VMEM((_BB, _SEQ_LEN), jnp.int32),
            pltpu.VMEM((_SEQ_LEN, _XP), jnp.int32),
            pltpu.VMEM((2, _BB, _EMBED_DIM), jnp.float32),
            pltpu.VMEM((2, _DT, 8, _BB + 1), jnp.float32),
            pltpu.SemaphoreType.DMA((2,)),
            pltpu.SemaphoreType.DMA((2,)),
        ],
        compiler_params=pltpu.CompilerParams(
            use_tc_tiling_on_sc=False, needs_layout_passes=False
        ),
    )(x.astype(jnp.int32), tlin.reshape(_VOCAB, _EMBED_DIM))
    return o5.transpose(0, 2, 4, 1, 3).reshape(_SEQ_LEN, _BATCH, _EMBED_DIM)


# speedup vs baseline: 1.3867x; 1.3867x over previous
import jax
import jax.numpy as jnp
from jax import lax
from jax.experimental import pallas as pl
from jax.experimental.pallas import tpu as pltpu
from jax.experimental.pallas import tpu_sc as plsc

_EMBED_DIM = 32
_BATCH = 4096
_SEQ_LEN = 200
_VOCAB = 1000000

_NC = 2
_NS = 16
_NW = _NC * _NS
_BB = _BATCH // _NW
_DT = _EMBED_DIM // 8
_XP = 136

_NFULL = _VOCAB // 128
_BPW = 246
_ANB = 6
_TAIL_V = _NFULL * 128


def _body_a(tv_hbm, ttail_hbm, tl_hbm, vin, vmid, vout, vtail, vtout, gsem, wsem):
    w = lax.axis_index("s") * _NC + lax.axis_index("c")

    lane = lax.iota(jnp.int32, 16)
    dv = [(lane + 16 * jg) % _EMBED_DIM for jg in range(8)]
    sv = [(lane + 16 * jg) // _EMBED_DIM for jg in range(8)]
    vz = [(lane + 16 * vg) * 33 for vg in range(8)]
    bz = [sv[jg] * 33 + dv[jg] for jg in range(8)]

    def c0_of(i):
        return pl.multiple_of(
            jnp.minimum(w * _BPW + i, _NFULL - 1) * 128, 128
        )

    def fire_in(i, nb):
        pltpu.async_copy(
            tv_hbm.at[:, pl.ds(c0_of(i), 128)], vin.at[nb], gsem.at[nb]
        )

    def drain_in(nb):
        pltpu.make_async_copy(
            tv_hbm.at[:, pl.ds(0, 128)], vin.at[nb], gsem.at[nb]
        ).wait()

    def fire_out(i, nb):
        pltpu.async_copy(
            vout.at[nb],
            tl_hbm.at[pl.ds(pl.multiple_of(c0_of(i) // 4, 8), 32)],
            wsem.at[nb],
        )

    def drain_out(nb):
        pltpu.make_async_copy(
            vout.at[nb], tl_hbm.at[pl.ds(0, 32)], wsem.at[nb]
        ).wait()

    def transpose(nb):
        @pl.loop(0, _EMBED_DIM)
        def _scat(d):
            for vg in range(8):
                v = vin[nb, d, pl.ds(16 * vg, 16)]
                plsc.store_scatter(vmid, [vz[vg] + d], v)

        @pl.loop(0, 32)
        def _row(r):
            for jg in range(8):
                v = plsc.load_gather(vmid, [bz[jg] + 132 * r])
                vout[nb, r, pl.ds(16 * jg, 16)] = v

    for b in range(_ANB):
        fire_in(b, b)
    for b in range(_ANB):
        drain_in(b)
        transpose(b)
        fire_in(b + _ANB, b)
        fire_out(b, b)

    @pl.loop(1, _BPW // _ANB - 1)
    def _grp(g):
        i0 = g * _ANB
        for b in range(_ANB):
            i = i0 + b
            drain_in(b)
            drain_out(b)
            transpose(b)
            fire_in(i + _ANB, b)
            fire_out(i, b)

    for b in range(_ANB):
        i = _BPW - _ANB + b
        drain_in(b)
        drain_out(b)
        transpose(b)
        fire_out(i, b)
    for b in range(_ANB):
        drain_out(b)

    @pl.when(w == 0)
    def _tail():
        pltpu.sync_copy(ttail_hbm, vtail)
        for r in range(16):
            for jg in range(8):
                v = plsc.load_gather(vtail, [sv[jg] + 4 * r, dv[jg]])
                vtout[r, pl.ds(16 * jg, 16)] = v
        pltpu.sync_copy(vtout, tl_hbm.at[pl.ds(_TAIL_V // 4, 16)])


def _body_b(x_hbm, table_hbm, o5_hbm, xblk, xt, rows_v, t_v, gsem, wsem):
    w = lax.axis_index("s") * _NC + lax.axis_index("c")

    pltpu.sync_copy(x_hbm.at[pl.ds(w * _BB, _BB)], xblk)

    lane = lax.iota(jnp.int32, 16)
    nh = (_SEQ_LEN + 15) // 16
    l_off = [16 * h if 16 * h + 16 <= _SEQ_LEN else _SEQ_LEN - 16
             for h in range(nh)]

    @pl.loop(0, _BB)
    def _xpose(b):
        bi = jnp.full((16,), 1, jnp.int32) * b
        for h in range(nh):
            v = xblk[b, pl.ds(l_off[h], 16)]
            plsc.store_scatter(xt, [lane + l_off[h], bi], v)

    dt_idx = [(lane + 16 * h) // 8 for h in range(2)]
    di_idx = [(lane + 16 * h) % 8 for h in range(2)]

    def fire_gather(l, nb):
        pltpu.async_copy(
            table_hbm.at[xt.at[l, pl.ds(0, _BB)]],
            rows_v.at[nb],
            gsem.at[nb],
        )

    def drain_gather(nb):
        pltpu.make_async_copy(
            table_hbm.at[pl.ds(0, _BB)], rows_v.at[nb], gsem.at[nb]
        ).wait()

    def fire_write(l, nb):
        pltpu.async_copy(
            t_v.at[nb].at[:, :, pl.ds(0, _BB)], o5_hbm.at[l].at[:, w],
            wsem.at[nb],
        )

    def drain_write(nb):
        pltpu.make_async_copy(
            t_v.at[nb].at[:, :, pl.ds(0, _BB)], o5_hbm.at[0].at[:, 0],
            wsem.at[nb],
        ).wait()

    def transpose(nb):
        @pl.loop(0, _BB)
        def _tpose(b):
            bi = jnp.full((16,), 1, jnp.int32) * b
            for h in range(2):
                v = rows_v[nb, b, pl.ds(16 * h, 16)]
                plsc.store_scatter(t_v.at[nb], [dt_idx[h], di_idx[h], bi], v)

    fire_gather(0, 0)
    fire_gather(1, 1)
    for b in range(2):
        drain_gather(b)
        transpose(b)
        fire_gather(b + 2, b)
        fire_write(b, b)

    @pl.loop(1, (_SEQ_LEN - 4) // 2 + 1)
    def _pair(p):
        l0 = 2 * p
        for b in range(2):
            l = l0 + b
            drain_gather(b)
            drain_write(b)
            transpose(b)
            fire_gather(l + 2, b)
            fire_write(l, b)

    for b in range(2):
        l = _SEQ_LEN - 2 + b
        drain_gather(b)
        drain_write(b)
        transpose(b)
        fire_write(l, b)
    for b in range(2):
        drain_write(b)


@jax.jit
def kernel(x, table):
    mesh = plsc.VectorSubcoreMesh(
        core_axis_name="c", subcore_axis_name="s",
        num_cores=_NC, num_subcores=_NS,
    )

    tlin = pl.kernel(
        _body_a,
        out_type=jax.ShapeDtypeStruct((_VOCAB // 4, 128), jnp.float32),
        mesh=mesh,
        scratch_types=[
            pltpu.VMEM((_ANB, _EMBED_DIM, 128), jnp.float32),
            pltpu.VMEM((128 * 33,), jnp.float32),
            pltpu.VMEM((_ANB, _EMBED_DIM, 128), jnp.float32),
            pltpu.VMEM((_VOCAB - _TAIL_V, _EMBED_DIM), jnp.float32),
            pltpu.VMEM(((_VOCAB - _TAIL_V) // 4, 128), jnp.float32),
            pltpu.SemaphoreType.DMA((2,)),
            pltpu.SemaphoreType.DMA((2,)),
        ],
        compiler_params=pltpu.CompilerParams(
            use_tc_tiling_on_sc=True, needs_layout_passes=False
        ),
    )(jnp.transpose(table),
      lax.slice(table, (_TAIL_V, 0), (_VOCAB, _EMBED_DIM)))

    o5 = pl.kernel(
        _body_b,
        out_type=jax.ShapeDtypeStruct(
            (_SEQ_LEN, _DT, _NW, 8, _BB), jnp.float32
        ),
        mesh=mesh,
        scratch_types=[
            pltpu.VMEM((_BB, _SEQ_LEN), jnp.int32),
            pltpu.VMEM((_SEQ_LEN, _XP), jnp.int32),
            pltpu.VMEM((2, _BB, _EMBED_DIM), jnp.float32),
            pltpu.VMEM((2, _DT, 8, _BB + 1), jnp.float32),
            pltpu.SemaphoreType.DMA((2,)),
            pltpu.SemaphoreType.DMA((2,)),
        ],
        compiler_params=pltpu.CompilerParams(
            use_tc_tiling_on_sc=False, needs_layout_passes=False
        ),
    )(x.astype(jnp.int32), tlin.reshape(_VOCAB, _EMBED_DIM))
    return o5.transpose(0, 2, 4, 1, 3).reshape(_SEQ_LEN, _BATCH, _EMBED_DIM)


# speedup vs baseline: 1.4157x; 1.0210x over previous
import jax
import jax.numpy as jnp
from jax import lax
from jax.experimental import pallas as pl
from jax.experimental.pallas import tpu as pltpu
from jax.experimental.pallas import tpu_sc as plsc

_EMBED_DIM = 32
_BATCH = 4096
_SEQ_LEN = 200
_VOCAB = 1000000

_NC = 2
_NS = 16
_NW = _NC * _NS
_BB = _BATCH // _NW
_DT = _EMBED_DIM // 8
_XP = 136

_NFULL = _VOCAB // 128
_BPW = 246
_ANB = 6
_TAIL_V = _NFULL * 128


def _body_a(tv_hbm, ttail_hbm, tl_hbm, vin, vmid, vout, vtail, vtout, gsem, wsem):
    w = lax.axis_index("s") * _NC + lax.axis_index("c")

    lane = lax.iota(jnp.int32, 16)
    dv = [(lane + 16 * jg) % _EMBED_DIM for jg in range(8)]
    sv = [(lane + 16 * jg) // _EMBED_DIM for jg in range(8)]
    vz = [(lane + 16 * vg) * 33 for vg in range(8)]
    bz = [sv[jg] * 33 + dv[jg] for jg in range(8)]

    def c0_of(i):
        return pl.multiple_of(
            jnp.minimum(w * _BPW + i, _NFULL - 1) * 128, 128
        )

    def fire_in(i, nb):
        pltpu.async_copy(
            tv_hbm.at[:, pl.ds(c0_of(i), 128)], vin.at[nb], gsem.at[nb]
        )

    def drain_in(nb):
        pltpu.make_async_copy(
            tv_hbm.at[:, pl.ds(0, 128)], vin.at[nb], gsem.at[nb]
        ).wait()

    def fire_out(i, nb):
        pltpu.async_copy(
            vout.at[nb],
            tl_hbm.at[pl.ds(pl.multiple_of(c0_of(i) // 4, 8), 32)],
            wsem.at[nb],
        )

    def drain_out(nb):
        pltpu.make_async_copy(
            vout.at[nb], tl_hbm.at[pl.ds(0, 32)], wsem.at[nb]
        ).wait()

    def transpose(nb):
        @pl.loop(0, _EMBED_DIM, unroll=4)
        def _scat(d):
            for vg in range(8):
                v = vin[nb, d, pl.ds(16 * vg, 16)]
                plsc.store_scatter(vmid, [vz[vg] + d], v)

        @pl.loop(0, 32, unroll=4)
        def _row(r):
            for jg in range(8):
                v = plsc.load_gather(vmid, [bz[jg] + 132 * r])
                vout[nb, r, pl.ds(16 * jg, 16)] = v

    for b in range(_ANB):
        fire_in(b, b)
    for b in range(_ANB):
        drain_in(b)
        transpose(b)
        fire_in(b + _ANB, b)
        fire_out(b, b)

    @pl.loop(1, _BPW // _ANB - 1)
    def _grp(g):
        i0 = g * _ANB
        for b in range(_ANB):
            i = i0 + b
            drain_in(b)
            drain_out(b)
            transpose(b)
            fire_in(i + _ANB, b)
            fire_out(i, b)

    for b in range(_ANB):
        i = _BPW - _ANB + b
        drain_in(b)
        drain_out(b)
        transpose(b)
        fire_out(i, b)
    for b in range(_ANB):
        drain_out(b)

    @pl.when(w == 0)
    def _tail():
        pltpu.sync_copy(ttail_hbm, vtail)
        for r in range(16):
            for jg in range(8):
                v = plsc.load_gather(vtail, [sv[jg] + 4 * r, dv[jg]])
                vtout[r, pl.ds(16 * jg, 16)] = v
        pltpu.sync_copy(vtout, tl_hbm.at[pl.ds(_TAIL_V // 4, 16)])


def _body_b(x_hbm, table_hbm, o5_hbm, xblk, xt, rows_v, t_v, gsem, wsem):
    w = lax.axis_index("s") * _NC + lax.axis_index("c")

    pltpu.sync_copy(x_hbm.at[pl.ds(w * _BB, _BB)], xblk)

    lane = lax.iota(jnp.int32, 16)
    nh = (_SEQ_LEN + 15) // 16
    l_off = [16 * h if 16 * h + 16 <= _SEQ_LEN else _SEQ_LEN - 16
             for h in range(nh)]

    @pl.loop(0, _BB)
    def _xpose(b):
        bi = jnp.full((16,), 1, jnp.int32) * b
        for h in range(nh):
            v = xblk[b, pl.ds(l_off[h], 16)]
            plsc.store_scatter(xt, [lane + l_off[h], bi], v)

    dt_idx = [(lane + 16 * h) // 8 for h in range(2)]
    di_idx = [(lane + 16 * h) % 8 for h in range(2)]

    def fire_gather(l, nb):
        pltpu.async_copy(
            table_hbm.at[xt.at[l, pl.ds(0, _BB)]],
            rows_v.at[nb],
            gsem.at[nb],
        )

    def drain_gather(nb):
        pltpu.make_async_copy(
            table_hbm.at[pl.ds(0, _BB)], rows_v.at[nb], gsem.at[nb]
        ).wait()

    def fire_write(l, nb):
        pltpu.async_copy(
            t_v.at[nb].at[:, :, pl.ds(0, _BB)], o5_hbm.at[l].at[:, w],
            wsem.at[nb],
        )

    def drain_write(nb):
        pltpu.make_async_copy(
            t_v.at[nb].at[:, :, pl.ds(0, _BB)], o5_hbm.at[0].at[:, 0],
            wsem.at[nb],
        ).wait()

    def transpose(nb):
        @pl.loop(0, _BB, unroll=4)
        def _tpose(b):
            bi = jnp.full((16,), 1, jnp.int32) * b
            for h in range(2):
                v = rows_v[nb, b, pl.ds(16 * h, 16)]
                plsc.store_scatter(t_v.at[nb], [dt_idx[h], di_idx[h], bi], v)

    fire_gather(0, 0)
    fire_gather(1, 1)
    for b in range(2):
        drain_gather(b)
        transpose(b)
        fire_gather(b + 2, b)
        fire_write(b, b)

    @pl.loop(1, (_SEQ_LEN - 4) // 2 + 1)
    def _pair(p):
        l0 = 2 * p
        for b in range(2):
            l = l0 + b
            drain_gather(b)
            drain_write(b)
            transpose(b)
            fire_gather(l + 2, b)
            fire_write(l, b)

    for b in range(2):
        l = _SEQ_LEN - 2 + b
        drain_gather(b)
        drain_write(b)
        transpose(b)
        fire_write(l, b)
    for b in range(2):
        drain_write(b)


@jax.jit
def kernel(x, table):
    mesh = plsc.VectorSubcoreMesh(
        core_axis_name="c", subcore_axis_name="s",
        num_cores=_NC, num_subcores=_NS,
    )

    tlin = pl.kernel(
        _body_a,
        out_type=jax.ShapeDtypeStruct((_VOCAB // 4, 128), jnp.float32),
        mesh=mesh,
        scratch_types=[
            pltpu.VMEM((_ANB, _EMBED_DIM, 128), jnp.float32),
            pltpu.VMEM((128 * 33,), jnp.float32),
            pltpu.VMEM((_ANB, _EMBED_DIM, 128), jnp.float32),
            pltpu.VMEM((_VOCAB - _TAIL_V, _EMBED_DIM), jnp.float32),
            pltpu.VMEM(((_VOCAB - _TAIL_V) // 4, 128), jnp.float32),
            pltpu.SemaphoreType.DMA((2,)),
            pltpu.SemaphoreType.DMA((2,)),
        ],
        compiler_params=pltpu.CompilerParams(
            use_tc_tiling_on_sc=True, needs_layout_passes=False
        ),
    )(jnp.transpose(table),
      lax.slice(table, (_TAIL_V, 0), (_VOCAB, _EMBED_DIM)))

    o5 = pl.kernel(
        _body_b,
        out_type=jax.ShapeDtypeStruct(
            (_SEQ_LEN, _DT, _NW, 8, _BB), jnp.float32
        ),
        mesh=mesh,
        scratch_types=[
            pltpu.VMEM((_BB, _SEQ_LEN), jnp.int32),
            pltpu.VMEM((_SEQ_LEN, _XP), jnp.int32),
            pltpu.VMEM((2, _BB, _EMBED_DIM), jnp.float32),
            pltpu.VMEM((2, _DT, 8, _BB + 1), jnp.float32),
            pltpu.SemaphoreType.DMA((2,)),
            pltpu.SemaphoreType.DMA((2,)),
        ],
        compiler_params=pltpu.CompilerParams(
            use_tc_tiling_on_sc=False, needs_layout_passes=False
        ),
    )(x.astype(jnp.int32), tlin.reshape(_VOCAB, _EMBED_DIM))
    return o5.transpose(0, 2, 4, 1, 3).reshape(_SEQ_LEN, _BATCH, _EMBED_DIM)
